# baseline (speedup 1.0000x reference)
import functools
import math

import jax
import jax.numpy as jnp
from jax.experimental import pallas as pl
from jax.experimental.pallas import tpu as pltpu

KSZ = 5
HALF = 2
SIGMA = 2.0
ORIG_H, ORIG_W = 384, 288
NUM_HEADS = 8
EPS = 1e-5


def _kernel(idx_ref,
            fmap_ref,
            meta_ref,
            wq_ref, bq_ref, wk_ref, bk_ref, wv_ref, bv_ref, wo_ref, bo_ref,
            wg_ref, bg_ref, p1_ref, p1b_ref, lng_ref, lnb_ref, p2_ref, p2b_ref,
            out_ref,
            *, n_kpts, n_ch, width):
    b = pl.program_id(0)

    e = [math.exp(-((k - HALF) ** 2) / (2.0 * SIGMA ** 2)) for k in range(KSZ)]
    s1 = sum(e)

    lane = jax.lax.broadcasted_iota(jnp.int32, (1, width), 1)

    pooled_rows = []
    for n in range(n_kpts):
        xi = idx_ref[b, n, 0]
        yi = idx_ref[b, n, 1]
        rows = []
        for dy in range(KSZ):
            row = fmap_ref[0, :, pl.ds(yi - HALF + dy, 1), :]
            rows.append(jnp.reshape(row, (n_ch, width)))
        acc = (e[0] * (rows[0] + rows[4]) + e[1] * (rows[1] + rows[3])
               + e[2] * rows[2])
        d = (lane - xi).astype(jnp.float32)
        wx = jnp.where(jnp.abs(lane - xi) <= HALF,
                       jnp.exp(d * d * (-1.0 / (2.0 * SIGMA ** 2))), 0.0)
        wx = wx * (1.0 / (s1 * s1))
        pooled = jax.lax.dot_general(
            wx, acc, (((1,), (1,)), ((), ())),
            preferred_element_type=jnp.float32,
            precision=jax.lax.Precision.HIGHEST)
        pooled_rows.append(pooled)

    A = jnp.concatenate(pooled_rows, axis=0)
    M = meta_ref[...]

    f32 = jnp.float32
    dk = n_ch // NUM_HEADS
    ch_i = jax.lax.broadcasted_iota(jnp.int32, (n_ch, NUM_HEADS), 0)
    hd_i = jax.lax.broadcasted_iota(jnp.int32, (n_ch, NUM_HEADS), 1)
    Hm = (ch_i // dk == hd_i).astype(f32)

    def dot(x, w):
        return jax.lax.dot_general(x, w, (((1,), (0,)), ((), ())),
                                   preferred_element_type=f32,
                                   precision=jax.lax.Precision.HIGHEST)

    h = dot(A, p1_ref[:n_ch, :]) + dot(M, p1_ref[n_ch:, :]) + p1b_ref[...]
    mu = jnp.mean(h, axis=1, keepdims=True)
    var = jnp.mean((h - mu) ** 2, axis=1, keepdims=True)
    hn = (h - mu) * jax.lax.rsqrt(var + EPS) * lng_ref[...] + lnb_ref[...]
    hn = jnp.maximum(hn, 0.0)
    projected = dot(hn, p2_ref[...]) + p2b_ref[...]

    QA = dot(A, wq_ref[...]) + bq_ref[...]
    QM = dot(M, wq_ref[...]) + bq_ref[...]
    KA = dot(A, wk_ref[...]) + bk_ref[...]
    KM = dot(M, wk_ref[...]) + bk_ref[...]
    VA = dot(A, wv_ref[...]) + bv_ref[...]
    VM = dot(M, wv_ref[...]) + bv_ref[...]

    scale = 1.0 / math.sqrt(float(dk))
    sAA = dot(QA * KA, Hm) * scale
    sAM = dot(QA * KM, Hm) * scale
    sMA = dot(QM * KA, Hm) * scale
    sMM = dot(QM * KM, Hm) * scale

    def softmax2(s0, s1_):
        m = jnp.maximum(s0, s1_)
        p0 = jnp.exp(s0 - m)
        p1 = jnp.exp(s1_ - m)
        r = 1.0 / (p0 + p1)
        return p0 * r, p1 * r

    wAA, wAM = softmax2(sAA, sAM)
    wMA, wMM = softmax2(sMA, sMM)

    HmT = jnp.transpose(Hm)
    YA = dot(wAA, HmT) * VA + dot(wAM, HmT) * VM
    YM = dot(wMA, HmT) * VA + dot(wMM, HmT) * VM

    gA = jax.nn.sigmoid(dot(A, wg_ref[...]) + bg_ref[...])
    gM = jax.nn.sigmoid(dot(M, wg_ref[...]) + bg_ref[...])
    YA = YA * dot(gA, HmT)
    YM = YM * dot(gM, HmT)

    outA = dot(YA, wo_ref[...]) + bo_ref[...]
    outM = dot(YM, wo_ref[...]) + bo_ref[...]

    out_ref[0, :, :] = (outA + outM) * 0.5 + projected


def kernel(feature_map, keypoints, meta, Wq, bq, Wk, bk, Wv, bv, Wo, bo,
           Wg, bg, P1w, P1b, ln_g, ln_b, P2w, P2b, *, interpret=False):
    B, C, H, W = feature_map.shape
    N = keypoints.shape[1]

    scale = jnp.array([W / ORIG_W, H / ORIG_H], dtype=jnp.float32)
    kf = keypoints * scale
    xi = jnp.clip(kf[..., 0].astype(jnp.int32), HALF, W - HALF - 1)
    yi = jnp.clip(kf[..., 1].astype(jnp.int32), HALF, H - HALF - 1)
    idx = jnp.stack([xi, yi], axis=-1)

    def full(shape):
        return pl.BlockSpec(shape, lambda b, ref=None: tuple(0 for _ in shape))

    grid_spec = pltpu.PrefetchScalarGridSpec(
        num_scalar_prefetch=1,
        grid=(B,),
        in_specs=[
            pl.BlockSpec((1, C, H, W), lambda b, idx_ref: (b, 0, 0, 0)),
            full((N, C)),
            full((C, C)), full((C,)),
            full((C, C)), full((C,)),
            full((C, C)), full((C,)),
            full((C, C)), full((C,)),
            full((C, NUM_HEADS)), full((NUM_HEADS,)),
            full((2 * C, C)), full((C,)),
            full((C,)), full((C,)),
            full((C, C)), full((C,)),
        ],
        out_specs=pl.BlockSpec((1, N, C), lambda b, idx_ref: (b, 0, 0)),
    )

    fn = pl.pallas_call(
        functools.partial(_kernel, n_kpts=N, n_ch=C, width=W),
        grid_spec=grid_spec,
        out_shape=jax.ShapeDtypeStruct((B, N, C), jnp.float32),
        compiler_params=pltpu.CompilerParams(
            dimension_semantics=("parallel",),
            vmem_limit_bytes=48 * 1024 * 1024,
        ),
        interpret=interpret,
    )
    return fn(idx, feature_map, meta, Wq, bq, Wk, bk, Wv, bv, Wo, bo,
              Wg, bg, P1w, P1b, ln_g, ln_b, P2w, P2b)


# speedup vs baseline: 1.8036x; 1.8036x over previous
import functools
import math

import jax
import jax.numpy as jnp
from jax.experimental import pallas as pl
from jax.experimental.pallas import tpu as pltpu

KSZ = 5
HALF = 2
SIGMA = 2.0
ORIG_H, ORIG_W = 384, 288
NUM_HEADS = 8
EPS = 1e-5


def _kernel(xyf_ref,
            fmap_ref,
            cst_ref,
            meta_ref,
            wq_ref, bq_ref, wk_ref, bk_ref, wv_ref, bv_ref, wo_ref, bo_ref,
            wg_ref, bg_ref, p1_ref, p1b_ref, lng_ref, lnb_ref, p2_ref, p2b_ref,
            out_ref,
            *, n_kpts, n_ch):
    f32 = jnp.float32

    e1 = [math.exp(-((k - HALF) ** 2) / (2.0 * SIGMA ** 2)) for k in range(KSZ)]
    inv_norm = 1.0 / (sum(e1) ** 2)

    xy = xyf_ref[0]
    xb = xy[:, 0:1]
    yb = xy[:, 1:2]
    YY = cst_ref[0:1, :]
    WW = cst_ref[1:2, :]
    dy = YY - yb
    dx = WW - xb
    g = jnp.exp((dy * dy + dx * dx) * (-1.0 / (2.0 * SIGMA ** 2))) * inv_norm
    inside = (jnp.abs(dy) <= float(HALF)) & (jnp.abs(dx) <= float(HALF))
    stamp = jnp.where(inside, g, 0.0)

    AT = jax.lax.dot_general(fmap_ref[0], stamp, (((1,), (1,)), ((), ())),
                             preferred_element_type=f32)
    A = jnp.transpose(AT)
    M = meta_ref[...]

    dk = n_ch // NUM_HEADS
    ch_i = jax.lax.broadcasted_iota(jnp.int32, (n_ch, NUM_HEADS), 0)
    hd_i = jax.lax.broadcasted_iota(jnp.int32, (n_ch, NUM_HEADS), 1)
    Hm = (ch_i // dk == hd_i).astype(f32)

    def dot(x, w):
        return jax.lax.dot_general(x, w, (((1,), (0,)), ((), ())),
                                   preferred_element_type=f32,
                                   precision=jax.lax.Precision.HIGHEST)

    h = dot(A, p1_ref[:n_ch, :]) + dot(M, p1_ref[n_ch:, :]) + p1b_ref[...]
    mu = jnp.mean(h, axis=1, keepdims=True)
    var = jnp.mean((h - mu) ** 2, axis=1, keepdims=True)
    hn = (h - mu) * jax.lax.rsqrt(var + EPS) * lng_ref[...] + lnb_ref[...]
    hn = jnp.maximum(hn, 0.0)
    projected = dot(hn, p2_ref[...]) + p2b_ref[...]

    QA = dot(A, wq_ref[...]) + bq_ref[...]
    QM = dot(M, wq_ref[...]) + bq_ref[...]
    KA = dot(A, wk_ref[...]) + bk_ref[...]
    KM = dot(M, wk_ref[...]) + bk_ref[...]
    VA = dot(A, wv_ref[...]) + bv_ref[...]
    VM = dot(M, wv_ref[...]) + bv_ref[...]

    scale = 1.0 / math.sqrt(float(dk))
    sAA = dot(QA * KA, Hm) * scale
    sAM = dot(QA * KM, Hm) * scale
    sMA = dot(QM * KA, Hm) * scale
    sMM = dot(QM * KM, Hm) * scale

    def softmax2(s0, s1_):
        m = jnp.maximum(s0, s1_)
        p0 = jnp.exp(s0 - m)
        p1 = jnp.exp(s1_ - m)
        r = 1.0 / (p0 + p1)
        return p0 * r, p1 * r

    wAA, wAM = softmax2(sAA, sAM)
    wMA, wMM = softmax2(sMA, sMM)

    HmT = jnp.transpose(Hm)
    YA = dot(wAA, HmT) * VA + dot(wAM, HmT) * VM
    YM = dot(wMA, HmT) * VA + dot(wMM, HmT) * VM

    gA = jax.nn.sigmoid(dot(A, wg_ref[...]) + bg_ref[...])
    gM = jax.nn.sigmoid(dot(M, wg_ref[...]) + bg_ref[...])
    YA = YA * dot(gA, HmT)
    YM = YM * dot(gM, HmT)

    outA = dot(YA, wo_ref[...]) + bo_ref[...]
    outM = dot(YM, wo_ref[...]) + bo_ref[...]

    out_ref[0, :, :] = (outA + outM) * 0.5 + projected


def kernel(feature_map, keypoints, meta, Wq, bq, Wk, bk, Wv, bv, Wo, bo,
           Wg, bg, P1w, P1b, ln_g, ln_b, P2w, P2b, *, interpret=False):
    B, C, H, W = feature_map.shape
    N = keypoints.shape[1]
    HW = H * W

    scale = jnp.array([W / ORIG_W, H / ORIG_H], dtype=jnp.float32)
    kf = keypoints * scale
    xi = jnp.clip(jnp.floor(kf[..., 0]), HALF, W - HALF - 1)
    yi = jnp.clip(jnp.floor(kf[..., 1]), HALF, H - HALF - 1)
    xyf = jnp.stack([xi, yi], axis=-1).astype(jnp.float32)

    flat = jnp.arange(HW, dtype=jnp.int32)
    cst = jnp.stack([flat // W, flat % W]).astype(jnp.float32)

    fm2 = feature_map.reshape(B, C, HW)

    def full(shape):
        return pl.BlockSpec(shape, lambda b: tuple(0 for _ in shape))

    fn = pl.pallas_call(
        functools.partial(_kernel, n_kpts=N, n_ch=C),
        grid=(B,),
        in_specs=[
            pl.BlockSpec((1, N, 2), lambda b: (b, 0, 0)),
            pl.BlockSpec((1, C, HW), lambda b: (b, 0, 0)),
            full((2, HW)),
            full((N, C)),
            full((C, C)), full((C,)),
            full((C, C)), full((C,)),
            full((C, C)), full((C,)),
            full((C, C)), full((C,)),
            full((C, NUM_HEADS)), full((NUM_HEADS,)),
            full((2 * C, C)), full((C,)),
            full((C,)), full((C,)),
            full((C, C)), full((C,)),
        ],
        out_specs=pl.BlockSpec((1, N, C), lambda b: (b, 0, 0)),
        out_shape=jax.ShapeDtypeStruct((B, N, C), jnp.float32),
        compiler_params=pltpu.CompilerParams(
            dimension_semantics=("arbitrary",),
            vmem_limit_bytes=44 * 1024 * 1024,
        ),
        interpret=interpret,
    )
    return fn(xyf, fm2, cst, meta, Wq, bq, Wk, bk, Wv, bv, Wo, bo,
              Wg, bg, P1w, P1b, ln_g, ln_b, P2w, P2b)
